# baseline (speedup 1.0000x reference)
import functools

import jax
import jax.numpy as jnp
from jax import lax
from jax.experimental import pallas as pl
from jax.experimental.pallas import tpu as pltpu
from jax.experimental.pallas import tpu_sc as plsc

_LANES = 16
_NC, _NS = 2, 16
_NW = _NC * _NS
_CHUNK = 8
_DEPTH = 6


def _sc_body(x_hbm, pos_hbm, table_hbm, out_hbm, idx_v, *rest):
    in_bufs = rest[:_DEPTH]
    pe_bufs = rest[_DEPTH:2 * _DEPTH]
    gsem, lsem, ssem = rest[2 * _DEPTH:]
    wid = lax.axis_index("s") * _NC + lax.axis_index("c")
    n_chunks = pos_hbm.shape[1]
    d = x_hbm.shape[1]

    def rows(c):
        return pl.ds((wid * n_chunks + c) * _CHUNK, _CHUNK)

    pltpu.sync_copy(pos_hbm.at[wid], idx_v)

    gathers = [None] * n_chunks
    loads = [None] * n_chunks
    stores = [None] * n_chunks

    def prefetch(p):
        gathers[p] = pltpu.async_copy(
            table_hbm.at[idx_v.at[p]], pe_bufs[p % _DEPTH], gsem)
        loads[p] = pltpu.async_copy(
            x_hbm.at[rows(p)], in_bufs[p % _DEPTH], lsem)

    for p in range(_DEPTH - 1):
        prefetch(p)

    for c in range(n_chunks):
        p = c + _DEPTH - 1
        if p < n_chunks:
            if p - _DEPTH >= 0:
                stores[p - _DEPTH].wait()
            prefetch(p)
        gathers[c].wait()
        loads[c].wait()

        in_b = in_bufs[c % _DEPTH]
        pe_b = pe_bufs[c % _DEPTH]

        @plsc.parallel_loop(0, _CHUNK)
        def _row(r):
            @plsc.parallel_loop(0, d, _LANES, unroll=8)
            def _col(jj):
                sl = pl.ds(jj, _LANES)
                plsc.addupdate(in_b.at[r, sl], pe_b[r, sl])

        stores[c] = pltpu.async_copy(in_b, out_hbm.at[rows(c)], ssem)

    for c in range(max(0, n_chunks - _DEPTH), n_chunks):
        stores[c].wait()


@functools.partial(jax.jit, static_argnames=())
def kernel(inputs, inputs_positions, pos_embedding):
    b, s, d = inputs.shape
    n = b * s
    if inputs_positions is None:
        inputs_positions = jnp.broadcast_to(
            jnp.arange(s, dtype=jnp.int32)[None, :], (b, s))
    n_chunks = n // (_NW * _CHUNK)
    x = inputs.reshape(n, d)
    pos = inputs_positions.astype(jnp.int32).reshape(_NW, n_chunks, _CHUNK)
    table = pos_embedding.reshape(pos_embedding.shape[1], d)
    mesh = plsc.VectorSubcoreMesh(
        core_axis_name="c", subcore_axis_name="s",
        num_cores=_NC, num_subcores=_NS)
    scratch = [pltpu.VMEM((n_chunks, _CHUNK), jnp.int32)]
    scratch += [pltpu.VMEM((_CHUNK, d), jnp.float32)] * (2 * _DEPTH)
    scratch += [pltpu.SemaphoreType.DMA] * 3
    out = pl.kernel(
        _sc_body,
        out_type=jax.ShapeDtypeStruct((n, d), jnp.float32),
        mesh=mesh,
        scratch_types=scratch,
    )(x, pos, table)
    return out.reshape(b, s, d)


# speedup vs baseline: 1.0222x; 1.0222x over previous
import functools

import jax
import jax.numpy as jnp
from jax import lax
from jax.experimental import pallas as pl
from jax.experimental.pallas import tpu as pltpu
from jax.experimental.pallas import tpu_sc as plsc

_LANES = 16
_NC, _NS = 2, 16
_NW = _NC * _NS
_CHUNK = 16
_DEPTH = 3


def _sc_body(x_hbm, pos_hbm, table_hbm, out_hbm, idx_v, *rest):
    in_bufs = rest[:_DEPTH]
    pe_bufs = rest[_DEPTH:2 * _DEPTH]
    gsem, lsem, ssem = rest[2 * _DEPTH:]
    wid = lax.axis_index("s") * _NC + lax.axis_index("c")
    n_chunks = pos_hbm.shape[1]
    d = x_hbm.shape[1]

    def rows(c):
        return pl.ds((wid * n_chunks + c) * _CHUNK, _CHUNK)

    pltpu.sync_copy(pos_hbm.at[wid], idx_v)

    gathers = [None] * n_chunks
    loads = [None] * n_chunks
    stores = [None] * n_chunks

    def prefetch(p):
        gathers[p] = pltpu.async_copy(
            table_hbm.at[idx_v.at[p]], pe_bufs[p % _DEPTH], gsem)
        loads[p] = pltpu.async_copy(
            x_hbm.at[rows(p)], in_bufs[p % _DEPTH], lsem)

    for p in range(_DEPTH - 1):
        prefetch(p)

    for c in range(n_chunks):
        p = c + _DEPTH - 1
        if p < n_chunks:
            if p - _DEPTH >= 0:
                stores[p - _DEPTH].wait()
            prefetch(p)
        gathers[c].wait()
        loads[c].wait()

        in_b = in_bufs[c % _DEPTH]
        pe_b = pe_bufs[c % _DEPTH]

        @plsc.parallel_loop(0, _CHUNK)
        def _row(r):
            @plsc.parallel_loop(0, d, _LANES, unroll=16)
            def _col(jj):
                sl = pl.ds(jj, _LANES)
                plsc.addupdate(in_b.at[r, sl], pe_b[r, sl])

        stores[c] = pltpu.async_copy(in_b, out_hbm.at[rows(c)], ssem)

    for c in range(max(0, n_chunks - _DEPTH), n_chunks):
        stores[c].wait()


@functools.partial(jax.jit, static_argnames=())
def kernel(inputs, inputs_positions, pos_embedding):
    b, s, d = inputs.shape
    n = b * s
    if inputs_positions is None:
        inputs_positions = jnp.broadcast_to(
            jnp.arange(s, dtype=jnp.int32)[None, :], (b, s))
    n_chunks = n // (_NW * _CHUNK)
    x = inputs.reshape(n, d)
    pos = inputs_positions.astype(jnp.int32).reshape(_NW, n_chunks, _CHUNK)
    table = pos_embedding.reshape(pos_embedding.shape[1], d)
    mesh = plsc.VectorSubcoreMesh(
        core_axis_name="c", subcore_axis_name="s",
        num_cores=_NC, num_subcores=_NS)
    scratch = [pltpu.VMEM((n_chunks, _CHUNK), jnp.int32)]
    scratch += [pltpu.VMEM((_CHUNK, d), jnp.float32)] * (2 * _DEPTH)
    scratch += [pltpu.SemaphoreType.DMA] * 3
    out = pl.kernel(
        _sc_body,
        out_type=jax.ShapeDtypeStruct((n, d), jnp.float32),
        mesh=mesh,
        scratch_types=scratch,
    )(x, pos, table)
    return out.reshape(b, s, d)


# speedup vs baseline: 1.0436x; 1.0210x over previous
import functools

import jax
import jax.numpy as jnp
from jax import lax
from jax.experimental import pallas as pl
from jax.experimental.pallas import tpu as pltpu
from jax.experimental.pallas import tpu_sc as plsc

_LANES = 16
_NC, _NS = 2, 16
_NW = _NC * _NS
_CHUNK = 16
_DEPTH = 3


def _sc_body(x_hbm, pos_hbm, table_hbm, out_hbm, idx_v, *rest):
    in_bufs = rest[:_DEPTH]
    pe_bufs = rest[_DEPTH:2 * _DEPTH]
    gsem, lsem, ssem = rest[2 * _DEPTH:]
    wid = lax.axis_index("s") * _NC + lax.axis_index("c")
    n_chunks = pos_hbm.shape[1]
    d = x_hbm.shape[1]

    def rows(c):
        return pl.ds((wid * n_chunks + c) * _CHUNK, _CHUNK)

    pltpu.sync_copy(pos_hbm.at[wid], idx_v)

    gathers = [None] * n_chunks
    loads = [None] * n_chunks
    stores = [None] * n_chunks

    def prefetch(p):
        gathers[p] = pltpu.async_copy(
            table_hbm.at[idx_v.at[p]], pe_bufs[p % _DEPTH], gsem)
        loads[p] = pltpu.async_copy(
            x_hbm.at[rows(p)], in_bufs[p % _DEPTH], lsem)

    for p in range(_DEPTH - 1):
        prefetch(p)

    for c in range(n_chunks):
        p = c + _DEPTH - 1
        if p < n_chunks:
            if p - _DEPTH >= 0:
                stores[p - _DEPTH].wait()
            prefetch(p)
        gathers[c].wait()
        loads[c].wait()

        in_b = in_bufs[c % _DEPTH]
        pe_b = pe_bufs[c % _DEPTH]

        @plsc.parallel_loop(0, _CHUNK)
        def _row(r):
            @plsc.parallel_loop(0, d, _LANES, unroll=8)
            def _col(jj):
                sl = pl.ds(jj, _LANES)
                plsc.addupdate(in_b.at[r, sl], pe_b[r, sl])

        stores[c] = pltpu.async_copy(in_b, out_hbm.at[rows(c)], ssem)

    for c in range(max(0, n_chunks - _DEPTH), n_chunks):
        stores[c].wait()


@functools.partial(jax.jit, static_argnames=())
def kernel(inputs, inputs_positions, pos_embedding):
    b, s, d = inputs.shape
    n = b * s
    if inputs_positions is None:
        inputs_positions = jnp.broadcast_to(
            jnp.arange(s, dtype=jnp.int32)[None, :], (b, s))
    n_chunks = n // (_NW * _CHUNK)
    x = inputs.reshape(n, d)
    pos = inputs_positions.astype(jnp.int32).reshape(_NW, n_chunks, _CHUNK)
    table = pos_embedding.reshape(pos_embedding.shape[1], d)
    mesh = plsc.VectorSubcoreMesh(
        core_axis_name="c", subcore_axis_name="s",
        num_cores=_NC, num_subcores=_NS)
    scratch = [pltpu.VMEM((n_chunks, _CHUNK), jnp.int32)]
    scratch += [pltpu.VMEM((_CHUNK, d), jnp.float32)] * (2 * _DEPTH)
    scratch += [pltpu.SemaphoreType.DMA] * 3
    out = pl.kernel(
        _sc_body,
        out_type=jax.ShapeDtypeStruct((n, d), jnp.float32),
        mesh=mesh,
        scratch_types=scratch,
    )(x, pos, table)
    return out.reshape(b, s, d)
